# baseline (speedup 1.0000x reference)
import functools

import jax
import jax.numpy as jnp
from jax import lax
from jax.experimental import pallas as pl
from jax.experimental.pallas import tpu as pltpu
from jax.experimental.pallas import tpu_sc as plsc

_B = 16384
_D = 64
_LAT = 128
_NC = 2
_NS = 16
_NW = _NC * _NS
_BPW = _B // _NW
_CHUNK = 128
_NCH = _BPW // _CHUNK
_NGB = 8


def _ncf_body(uidx_hbm, iidx_hbm, uT_hbm, iT_hbm, uA_hbm, iA_hbm, par_hbm,
              out_hbm,
              uidx_v, iidx_v, urows_v, irows_v, uA_v, iA_v, par_v,
              wu_v, wi_v, out_v, sem):
    wid = lax.axis_index("s") * _NC + lax.axis_index("c")
    base = wid * _BPW

    for j in range(_NCH):
        pltpu.sync_copy(uidx_hbm.at[pl.ds(base + j * _CHUNK, _CHUNK)],
                        uidx_v.at[j])
        pltpu.sync_copy(iidx_hbm.at[pl.ds(base + j * _CHUNK, _CHUNK)],
                        iidx_v.at[j])

    copies = []
    for j in range(_NCH):
        copies.append(pltpu.async_copy(
            uT_hbm.at[uidx_v.at[j]], urows_v.at[pl.ds(j * _CHUNK, _CHUNK)],
            sem))
        copies.append(pltpu.async_copy(
            iT_hbm.at[iidx_v.at[j]], irows_v.at[pl.ds(j * _CHUNK, _CHUNK)],
            sem))

    pltpu.sync_copy(uA_hbm, uA_v)
    pltpu.sync_copy(iA_hbm, iA_v)
    pltpu.sync_copy(par_hbm, par_v)

    kidx = [lax.iota(jnp.int32, 16) + kc * 16 for kc in range(_D // 16)]

    def fold_body(l, carry):
        accs = list(carry)
        col = jnp.full((16,), l, dtype=jnp.int32)
        wlu = plsc.load_gather(par_v, [col])
        wli = plsc.load_gather(par_v, [col + _LAT])
        for kc in range(_D // 16):
            accs[kc] = accs[kc] + plsc.load_gather(uA_v, [kidx[kc], col]) * wlu
            accs[4 + kc] = (accs[4 + kc]
                            + plsc.load_gather(iA_v, [kidx[kc], col]) * wli)
        return tuple(accs)

    zero = jnp.zeros((16,), jnp.float32)
    accs = lax.fori_loop(0, _LAT, fold_body, (zero,) * 8)
    for kc in range(_D // 16):
        wu_v[pl.ds(kc * 16, 16)] = accs[kc]
        wi_v[pl.ds(kc * 16, 16)] = accs[4 + kc]

    for c in copies:
        c.wait()

    bias = plsc.load_gather(par_v, [jnp.full((16,), 2 * _LAT, jnp.int32)])
    iota16 = lax.iota(jnp.int32, 16)

    def dot_body(gb, carry):
        accs = [zero] * _NGB
        bidxs = [iota16 + (gb * _NGB + g) * 16 for g in range(_NGB)]
        for d in range(_D):
            col = jnp.full((16,), d, dtype=jnp.int32)
            wlu = plsc.load_gather(wu_v, [col])
            wli = plsc.load_gather(wi_v, [col])
            for g in range(_NGB):
                accs[g] = (accs[g]
                           + plsc.load_gather(urows_v, [bidxs[g], col]) * wlu
                           + plsc.load_gather(irows_v, [bidxs[g], col]) * wli)
        for g in range(_NGB):
            out_v[pl.ds((gb * _NGB + g) * 16, 16)] = accs[g] + bias
        return carry

    lax.fori_loop(0, _BPW // 16 // _NGB, dot_body, 0)

    pltpu.sync_copy(out_v, out_hbm.at[pl.ds(base, _BPW)])


@jax.jit
def _ncf(user_indices, item_indices, user_T, item_T, user_A, item_A, params):
    run = pl.kernel(
        _ncf_body,
        out_type=jax.ShapeDtypeStruct((_B,), jnp.float32),
        mesh=plsc.VectorSubcoreMesh(core_axis_name="c", subcore_axis_name="s"),
        compiler_params=pltpu.CompilerParams(needs_layout_passes=False,
                                             use_tc_tiling_on_sc=False),
        scratch_types=[
            pltpu.VMEM((_NCH, _CHUNK), jnp.int32),
            pltpu.VMEM((_NCH, _CHUNK), jnp.int32),
            pltpu.VMEM((_BPW, _D), jnp.float32),
            pltpu.VMEM((_BPW, _D), jnp.float32),
            pltpu.VMEM((_D, _LAT), jnp.float32),
            pltpu.VMEM((_D, _LAT), jnp.float32),
            pltpu.VMEM((264,), jnp.float32),
            pltpu.VMEM((_D,), jnp.float32),
            pltpu.VMEM((_D,), jnp.float32),
            pltpu.VMEM((_BPW,), jnp.float32),
            pltpu.SemaphoreType.DMA,
        ],
    )
    return run(user_indices, item_indices, user_T, item_T, user_A, item_A,
               params)


def kernel(user_indices, item_indices, user_T, item_T, user_A, item_A,
           W_aff, b_aff):
    params = jnp.concatenate([W_aff.reshape(-1), b_aff.reshape(-1),
                              jnp.zeros((7,), jnp.float32)])
    out = _ncf(user_indices.astype(jnp.int32), item_indices.astype(jnp.int32),
               user_T, item_T, user_A, item_A, params)
    return out.reshape(_B, 1)


# speedup vs baseline: 1.0009x; 1.0009x over previous
import functools

import jax
import jax.numpy as jnp
from jax import lax
from jax.experimental import pallas as pl
from jax.experimental.pallas import tpu as pltpu
from jax.experimental.pallas import tpu_sc as plsc

_B = 16384
_D = 64
_LAT = 128
_NC = 2
_NS = 16
_NW = _NC * _NS
_BPW = _B // _NW
_CHUNK = 128
_NCH = _BPW // _CHUNK
_NGB = 8
_NV2 = 500000


def _ncf_body(uidx_hbm, iidx_hbm, uT_hbm, iT_hbm, uA_hbm, iA_hbm, par_hbm,
              out_hbm,
              uidx_v, iidx_v, gidx_v, rows_v, uA_v, iA_v, par_v,
              wu_v, wi_v, out_v, sem):
    wid = lax.axis_index("s") * _NC + lax.axis_index("c")
    base = wid * _BPW

    for j in range(_NCH):
        pltpu.sync_copy(uidx_hbm.at[pl.ds(base + j * _CHUNK, _CHUNK)],
                        uidx_v.at[j])
        pltpu.sync_copy(iidx_hbm.at[pl.ds(base + j * _CHUNK, _CHUNK)],
                        iidx_v.at[j])

    for j in range(_NCH):
        for k in range(_CHUNK // 16):
            s = pl.ds(k * 16, 16)
            gidx_v[j, s] = lax.shift_right_logical(uidx_v[j, s], 1)

    copies = [pltpu.async_copy(
        uT_hbm.at[gidx_v.at[j]], rows_v.at[pl.ds(j * _CHUNK, _CHUNK)], sem)
        for j in range(_NCH)]

    pltpu.sync_copy(uA_hbm, uA_v)
    pltpu.sync_copy(iA_hbm, iA_v)
    pltpu.sync_copy(par_hbm, par_v)

    kidx = [lax.iota(jnp.int32, 16) + kc * 16 for kc in range(_D // 16)]

    def fold_body(l, carry):
        accs = list(carry)
        col = jnp.full((16,), l, dtype=jnp.int32)
        wlu = plsc.load_gather(par_v, [col])
        wli = plsc.load_gather(par_v, [col + _LAT])
        for kc in range(_D // 16):
            accs[kc] = accs[kc] + plsc.load_gather(uA_v, [kidx[kc], col]) * wlu
            accs[4 + kc] = (accs[4 + kc]
                            + plsc.load_gather(iA_v, [kidx[kc], col]) * wli)
        return tuple(accs)

    zero = jnp.zeros((16,), jnp.float32)
    accs = lax.fori_loop(0, _LAT, fold_body, (zero,) * 8)
    for kc in range(_D // 16):
        wu_v[pl.ds(kc * 16, 16)] = accs[kc]
        wi_v[pl.ds(kc * 16, 16)] = accs[4 + kc]

    bias = plsc.load_gather(par_v, [jnp.full((16,), 2 * _LAT, jnp.int32)])
    iota16 = lax.iota(jnp.int32, 16)

    def make_dot(idx_ref, w_ref, first):
        def dot_body(gb, carry):
            cbase = []
            for g in range(_NGB):
                e = gb * _NGB + g
                vi = idx_ref[e * 16 // _CHUNK, pl.ds((e * 16) % _CHUNK, 16)]
                cbase.append((vi & 1) * 64)
            accs = [zero] * _NGB
            bidxs = [iota16 + (gb * _NGB + g) * 16 for g in range(_NGB)]
            for d in range(_D):
                col = jnp.full((16,), d, dtype=jnp.int32)
                wl = plsc.load_gather(w_ref, [col])
                for g in range(_NGB):
                    accs[g] = accs[g] + plsc.load_gather(
                        rows_v, [bidxs[g], cbase[g] + col]) * wl
            for g in range(_NGB):
                s = pl.ds((gb * _NGB + g) * 16, 16)
                if first:
                    out_v[s] = accs[g] + bias
                else:
                    out_v[s] = out_v[s] + accs[g]
            return carry
        return dot_body

    for c in copies:
        c.wait()
    lax.fori_loop(0, _BPW // 16 // _NGB, make_dot(uidx_v, wu_v, True), 0)

    for j in range(_NCH):
        for k in range(_CHUNK // 16):
            s = pl.ds(k * 16, 16)
            gidx_v[j, s] = lax.shift_right_logical(iidx_v[j, s], 1)
    copies = [pltpu.async_copy(
        iT_hbm.at[gidx_v.at[j]], rows_v.at[pl.ds(j * _CHUNK, _CHUNK)], sem)
        for j in range(_NCH)]
    for c in copies:
        c.wait()
    lax.fori_loop(0, _BPW // 16 // _NGB, make_dot(iidx_v, wi_v, False), 0)

    pltpu.sync_copy(out_v, out_hbm.at[pl.ds(base, _BPW)])


@jax.jit
def _ncf(user_indices, item_indices, user_T2, item_T2, user_A, item_A,
         params):
    run = pl.kernel(
        _ncf_body,
        out_type=jax.ShapeDtypeStruct((_B,), jnp.float32),
        mesh=plsc.VectorSubcoreMesh(core_axis_name="c", subcore_axis_name="s"),
        compiler_params=pltpu.CompilerParams(needs_layout_passes=False,
                                             use_tc_tiling_on_sc=False),
        scratch_types=[
            pltpu.VMEM((_NCH, _CHUNK), jnp.int32),
            pltpu.VMEM((_NCH, _CHUNK), jnp.int32),
            pltpu.VMEM((_NCH, _CHUNK), jnp.int32),
            pltpu.VMEM((_BPW, _LAT), jnp.float32),
            pltpu.VMEM((_D, _LAT), jnp.float32),
            pltpu.VMEM((_D, _LAT), jnp.float32),
            pltpu.VMEM((264,), jnp.float32),
            pltpu.VMEM((_D,), jnp.float32),
            pltpu.VMEM((_D,), jnp.float32),
            pltpu.VMEM((_BPW,), jnp.float32),
            pltpu.SemaphoreType.DMA,
        ],
    )
    return run(user_indices, item_indices, user_T2, item_T2, user_A, item_A,
               params)


def kernel(user_indices, item_indices, user_T, item_T, user_A, item_A,
           W_aff, b_aff):
    params = jnp.concatenate([W_aff.reshape(-1), b_aff.reshape(-1),
                              jnp.zeros((7,), jnp.float32)])
    out = _ncf(user_indices.astype(jnp.int32), item_indices.astype(jnp.int32),
               user_T.reshape(_NV2, _LAT), item_T.reshape(_NV2, _LAT),
               user_A, item_A, params)
    return out.reshape(_B, 1)
